# Initial kernel scaffold; baseline (speedup 1.0000x reference)
import jax
import jax.numpy as jnp
from jax.experimental import pallas as pl


def kernel(x, edge_index, edge_weight, batch, W1, b1, g1, be1, W2, b2, g2, be2, W3, b3, g3, be3, g4, be4, Wl, bl):
    raise NotImplementedError("write your pallas kernel here")


# speedup vs baseline: 2.0412x; 2.0412x over previous
import dataclasses
import functools

import jax
import jax.numpy as jnp
from jax import lax
from jax.experimental import pallas as pl
from jax.experimental.pallas import tpu as pltpu
from jax.experimental.pallas import tpu_sc as plsc

N = 10000
N_PAD = 10240
E = 160000
F_IN = 256
H = 256
G = 64
D2 = H + F_IN
D3 = 2 * H + F_IN
CH = 128
BLK = 128
NSUB = 16
E_PAD = ((E + NSUB * BLK - 1) // (NSUB * BLK)) * NSUB * BLK
BPT = E_PAD // (NSUB * BLK)
RPT = N_PAD // NSUB
RB = 1280
NRB = N_PAD // RB
EPS = 1e-5


def _sc_agg(hw, gidx, dst_p, ew_p, nc):
    ncpc = nc // 2

    cp = pltpu.CompilerParams()
    if "needs_layout_passes" in pltpu.CompilerParams.__dataclass_fields__:
        cp = dataclasses.replace(cp, needs_layout_passes=False)

    @functools.partial(
        pl.kernel,
        compiler_params=cp,
        out_type=jax.ShapeDtypeStruct((nc * N_PAD, CH), jnp.float32),
        mesh=plsc.VectorSubcoreMesh(core_axis_name="c", subcore_axis_name="s"),
        scratch_types=[
            pltpu.VMEM((BLK,), jnp.int32),
            pltpu.VMEM((BLK,), jnp.int32),
            pltpu.VMEM((BLK,), jnp.float32),
            pltpu.VMEM((BLK, CH), jnp.float32),
            pltpu.VMEM_SHARED((N_PAD, CH), jnp.float32),
            pltpu.SemaphoreType.DMA,
        ],
    )
    def k(hw_hbm, gidx_hbm, dst_hbm, ew_hbm, out_hbm,
          idx_v, dst_v, ew_v, rows_v, acc_sh, sem):
        core = lax.axis_index("c")
        sid = lax.axis_index("s")
        zero = jnp.zeros((16,), jnp.float32)
        for ci in range(ncpc):
            chunk = core * ncpc + ci

            @pl.loop(0, BLK)
            def _zero_rows(r):
                for c0 in range(0, CH, 16):
                    rows_v[r, pl.ds(c0, 16)] = zero

            for z in range(5):
                pltpu.sync_copy(rows_v,
                                acc_sh.at[pl.ds(sid * RPT + z * BLK, BLK)])
            plsc.subcore_barrier()

            ebase = sid * (BPT * BLK)

            @pl.loop(0, BPT)
            def _edges(bi):
                eb = ebase + bi * BLK
                pltpu.sync_copy(gidx_hbm.at[pl.ds(chunk * E_PAD + eb, BLK)],
                                idx_v)
                pltpu.sync_copy(dst_hbm.at[pl.ds(eb, BLK)], dst_v)
                pltpu.sync_copy(ew_hbm.at[pl.ds(eb, BLK)], ew_v)
                pltpu.async_copy(hw_hbm.at[idx_v], rows_v, sem).wait()

                @pl.loop(0, BLK)
                def _scale(e):
                    w = plsc.load_gather(
                        ew_v, [jnp.zeros((16,), jnp.int32) + e])
                    for c0 in range(0, CH, 16):
                        rows_v[e, pl.ds(c0, 16)] = rows_v[e, pl.ds(c0, 16)] * w

                pltpu.sync_copy(rows_v, acc_sh.at[dst_v], add=True)

            plsc.subcore_barrier()
            for z in range(5):
                pltpu.sync_copy(acc_sh.at[pl.ds(sid * RPT + z * BLK, BLK)],
                                out_hbm.at[pl.ds(chunk * N_PAD + sid * RPT
                                                 + z * BLK, BLK)])
            if ci + 1 < ncpc:
                plsc.subcore_barrier()

    return k(hw, gidx, dst_p, ew_p)


_DNUMS = (((1,), (1,)), ((), ()))


def _tc_mm1(x, W1):
    nc = H // CH

    def body(x_ref, w_ref, o_ref):
        o_ref[...] = lax.dot_general(x_ref[...], w_ref[...], _DNUMS,
                                     preferred_element_type=jnp.float32)

    return pl.pallas_call(
        body,
        grid=(nc, NRB),
        in_specs=[pl.BlockSpec((RB, F_IN), lambda i, j: (j, 0)),
                  pl.BlockSpec((CH, F_IN), lambda i, j: (i, 0))],
        out_specs=pl.BlockSpec((RB, CH), lambda i, j: (i * NRB + j, 0)),
        out_shape=jax.ShapeDtypeStruct((nc * N_PAD, CH), jnp.float32),
    )(x, W1)


def _tc_stats(agg, b2d):
    nc = agg.shape[0]

    def body(a_ref, b_ref, o_ref):
        j = pl.program_id(0)

        @pl.when(j == 0)
        def _():
            o_ref[...] = jnp.zeros_like(o_ref)

        valid = (j * RB + lax.broadcasted_iota(jnp.int32, (RB, 1), 0)) < N
        for c in range(nc):
            t = jnp.maximum(a_ref[c] + b_ref[c][None, :], 0.0)
            t = jnp.where(valid, t, 0.0)
            o_ref[0, c] += jnp.sum(t, axis=0)
            o_ref[1, c] += jnp.sum(t * t, axis=0)

    return pl.pallas_call(
        body,
        grid=(NRB,),
        in_specs=[pl.BlockSpec((nc, RB, CH), lambda j: (0, j, 0)),
                  pl.BlockSpec((nc, CH), lambda j: (0, 0))],
        out_specs=pl.BlockSpec((2, nc, CH), lambda j: (0, 0, 0)),
        out_shape=jax.ShapeDtypeStruct((2, nc, CH), jnp.float32),
    )(agg, b2d)


def _affine(s, g2d, be2d):
    mean = s[0] / N
    var = s[1] / N - mean * mean
    a = g2d * lax.rsqrt(var + EPS)
    return a, be2d - mean * a


def _tc_mm2(agg1, b1r, a1, c1, x, W2h, W2x):
    nc_in = H // CH
    nc_out = D2 // CH

    def body(g_ref, b_ref, a_ref, c_ref, x_ref, wh_ref, wx_ref, o_ref):
        acc = lax.dot_general(x_ref[...], wx_ref[...], _DNUMS,
                              preferred_element_type=jnp.float32)
        for c in range(nc_in):
            t = jnp.maximum(g_ref[c] + b_ref[c][None, :], 0.0)
            u = a_ref[c][None, :] * t + c_ref[c][None, :]
            acc += lax.dot_general(u, wh_ref[:, c * CH:(c + 1) * CH], _DNUMS,
                                   preferred_element_type=jnp.float32)
        o_ref[...] = acc

    return pl.pallas_call(
        body,
        grid=(nc_out, NRB),
        in_specs=[pl.BlockSpec((nc_in, RB, CH), lambda i, j: (0, j, 0)),
                  pl.BlockSpec((nc_in, CH), lambda i, j: (0, 0)),
                  pl.BlockSpec((nc_in, CH), lambda i, j: (0, 0)),
                  pl.BlockSpec((nc_in, CH), lambda i, j: (0, 0)),
                  pl.BlockSpec((RB, F_IN), lambda i, j: (j, 0)),
                  pl.BlockSpec((CH, H), lambda i, j: (i, 0)),
                  pl.BlockSpec((CH, F_IN), lambda i, j: (i, 0))],
        out_specs=pl.BlockSpec((RB, CH), lambda i, j: (i * NRB + j, 0)),
        out_shape=jax.ShapeDtypeStruct((nc_out * N_PAD, CH), jnp.float32),
    )(agg1, b1r, a1, c1, x, W2h, W2x)


def _tc_mm3(agg2, b2r, a2, c2, agg1, b1r, a1, c1, W3h2, W3h1):
    nc2 = D2 // CH
    nc1 = H // CH
    nc_out = D3 // CH

    def body(g2_ref, b2_ref, a2_ref, c2_ref, g1_ref, b1_ref, a1_ref, c1_ref,
             wh2_ref, wh1_ref, o_ref):
        acc = None
        for c in range(nc2):
            t = jnp.maximum(g2_ref[c] + b2_ref[c][None, :], 0.0)
            u = a2_ref[c][None, :] * t + c2_ref[c][None, :]
            p = lax.dot_general(u, wh2_ref[:, c * CH:(c + 1) * CH], _DNUMS,
                                preferred_element_type=jnp.float32)
            acc = p if acc is None else acc + p
        for c in range(nc1):
            t = jnp.maximum(g1_ref[c] + b1_ref[c][None, :], 0.0)
            u = a1_ref[c][None, :] * t + c1_ref[c][None, :]
            acc += lax.dot_general(u, wh1_ref[:, c * CH:(c + 1) * CH], _DNUMS,
                                   preferred_element_type=jnp.float32)
        o_ref[...] = acc

    return pl.pallas_call(
        body,
        grid=(nc_out, NRB),
        in_specs=[pl.BlockSpec((nc2, RB, CH), lambda i, j: (0, j, 0)),
                  pl.BlockSpec((nc2, CH), lambda i, j: (0, 0)),
                  pl.BlockSpec((nc2, CH), lambda i, j: (0, 0)),
                  pl.BlockSpec((nc2, CH), lambda i, j: (0, 0)),
                  pl.BlockSpec((nc1, RB, CH), lambda i, j: (0, j, 0)),
                  pl.BlockSpec((nc1, CH), lambda i, j: (0, 0)),
                  pl.BlockSpec((nc1, CH), lambda i, j: (0, 0)),
                  pl.BlockSpec((nc1, CH), lambda i, j: (0, 0)),
                  pl.BlockSpec((CH, D2), lambda i, j: (i, 0)),
                  pl.BlockSpec((CH, H), lambda i, j: (i, 0))],
        out_specs=pl.BlockSpec((RB, CH), lambda i, j: (i * NRB + j, 0)),
        out_shape=jax.ShapeDtypeStruct((nc_out * N_PAD, CH), jnp.float32),
    )(agg2, b2r, a2, c2, agg1, b1r, a1, c1, W3h2, W3h1)


def _tc_pool(agg3, batch3d):
    nc = D3 // CH

    def body(a_ref, bt_ref, seg_ref, cnt_ref, s_ref):
        j = pl.program_id(0)

        @pl.when(j == 0)
        def _():
            seg_ref[...] = jnp.zeros_like(seg_ref)
            cnt_ref[...] = jnp.zeros_like(cnt_ref)
            s_ref[...] = jnp.zeros_like(s_ref)

        b = bt_ref[0, 0, :]
        oh = (b[:, None] == lax.broadcasted_iota(jnp.int32, (RB, G), 1)
              ).astype(jnp.float32)
        cnt_ref[0, :] += jnp.sum(oh, axis=0)
        for c in range(nc):
            blkc = a_ref[c]
            seg_ref[c] += lax.dot_general(oh, blkc, (((0,), (0,)), ((), ())),
                                          preferred_element_type=jnp.float32)
            s_ref[0, c] += jnp.sum(blkc, axis=0)
            s_ref[1, c] += jnp.sum(blkc * blkc, axis=0)

    return pl.pallas_call(
        body,
        grid=(NRB,),
        in_specs=[pl.BlockSpec((nc, RB, CH), lambda j: (0, j, 0)),
                  pl.BlockSpec((1, 1, RB), lambda j: (j, 0, 0))],
        out_specs=[pl.BlockSpec((nc, G, CH), lambda j: (0, 0, 0)),
                   pl.BlockSpec((1, G), lambda j: (0, 0)),
                   pl.BlockSpec((2, nc, CH), lambda j: (0, 0, 0))],
        out_shape=[jax.ShapeDtypeStruct((nc, G, CH), jnp.float32),
                   jax.ShapeDtypeStruct((1, G), jnp.float32),
                   jax.ShapeDtypeStruct((2, nc, CH), jnp.float32)],
    )(agg3, batch3d)


def _tc_final(seg, cnt, s3, b3r, g3r, be3r, g4r, be4r, wlr, bl2):
    nc = D3 // CH

    def body(seg_ref, cnt_ref, s_ref, b3_ref, g3_ref, be3_ref,
             g4_ref, be4_ref, wl_ref, bl_ref, o_ref):
        cnts = jnp.maximum(cnt_ref[0, :], 1.0)
        outv = jnp.zeros((G,), jnp.float32)
        for c in range(nc):
            m0 = s_ref[0, c] / N
            var3 = s_ref[1, c] / N - m0 * m0
            a3 = g3_ref[c] * lax.rsqrt(var3 + EPS)
            pooled = seg_ref[c] / cnts[:, None]
            p = a3[None, :] * (pooled - m0[None, :]) + be3_ref[c][None, :]
            m4 = jnp.mean(p, axis=0)
            v4 = jnp.mean(p * p, axis=0) - m4 * m4
            p4 = (g4_ref[c][None, :] * (p - m4[None, :])
                  * lax.rsqrt(v4 + EPS)[None, :] + be4_ref[c][None, :])
            outv += jnp.sum(p4 * wl_ref[c][None, :], axis=1)
        o_ref[...] = (outv + bl_ref[0, 0])[None, :]

    return pl.pallas_call(
        body,
        in_specs=[pl.BlockSpec(seg.shape, lambda: (0, 0, 0)),
                  pl.BlockSpec(cnt.shape, lambda: (0, 0)),
                  pl.BlockSpec(s3.shape, lambda: (0, 0, 0)),
                  pl.BlockSpec(b3r.shape, lambda: (0, 0)),
                  pl.BlockSpec(g3r.shape, lambda: (0, 0)),
                  pl.BlockSpec(be3r.shape, lambda: (0, 0)),
                  pl.BlockSpec(g4r.shape, lambda: (0, 0)),
                  pl.BlockSpec(be4r.shape, lambda: (0, 0)),
                  pl.BlockSpec(wlr.shape, lambda: (0, 0)),
                  pl.BlockSpec(bl2.shape, lambda: (0, 0))],
        out_specs=pl.BlockSpec((1, G), lambda: (0, 0)),
        out_shape=jax.ShapeDtypeStruct((1, G), jnp.float32),
    )(seg, cnt, s3, b3r, g3r, be3r, g4r, be4r, wlr, bl2)


def kernel(x, edge_index, edge_weight, batch, W1, b1, g1, be1, W2, b2, g2, be2,
           W3, b3, g3, be3, g4, be4, Wl, bl):
    f32 = jnp.float32
    x = x.astype(f32)
    src = edge_index[0].astype(jnp.int32)
    dst = edge_index[1].astype(jnp.int32)
    ew = edge_weight.astype(f32)
    pad = E_PAD - E
    src_p = jnp.concatenate([src, jnp.zeros((pad,), jnp.int32)])
    dst_p = jnp.concatenate([dst, jnp.zeros((pad,), jnp.int32)])
    ew_p = jnp.concatenate([ew, jnp.zeros((pad,), f32)])
    gidx6 = src_p[None, :] + (N_PAD * jnp.arange(6, dtype=jnp.int32))[:, None]
    x = jnp.concatenate([x, jnp.zeros((N_PAD - N, F_IN), f32)])
    batch_p = jnp.concatenate(
        [batch.astype(jnp.int32), jnp.full((N_PAD - N,), G, jnp.int32)])

    hw1 = _tc_mm1(x, W1)
    agg1 = _sc_agg(hw1, gidx6[:2].reshape(-1), dst_p, ew_p,
                   2).reshape(2, N_PAD, CH)
    b1r = b1.reshape(2, CH)
    a1, c1 = _affine(_tc_stats(agg1, b1r), g1.reshape(2, CH),
                     be1.reshape(2, CH))

    hw2 = _tc_mm2(agg1, b1r, a1, c1, x, W2[:, :H], W2[:, H:])
    agg2 = _sc_agg(hw2, gidx6[:4].reshape(-1), dst_p, ew_p,
                   4).reshape(4, N_PAD, CH)
    b2r = b2.reshape(4, CH)
    a2, c2 = _affine(_tc_stats(agg2, b2r), g2.reshape(4, CH),
                     be2.reshape(4, CH))

    hw3 = _tc_mm3(agg2, b2r, a2, c2, agg1, b1r, a1, c1, W3[:, :D2], W3[:, D2:])
    agg3 = _sc_agg(hw3, gidx6.reshape(-1), dst_p, ew_p,
                   6).reshape(6, N_PAD, CH)

    seg, cnt, s3 = _tc_pool(agg3, batch_p.reshape(NRB, 1, RB))
    out = _tc_final(seg, cnt, s3, b3.reshape(6, CH), g3.reshape(6, CH),
                    be3.reshape(6, CH), g4.reshape(6, CH), be4.reshape(6, CH),
                    Wl.reshape(6, CH), bl.reshape(1, 1))
    return out.reshape(G, 1)


# speedup vs baseline: 2.5523x; 1.2504x over previous
import dataclasses
import functools

import jax
import jax.numpy as jnp
from jax import lax
from jax.experimental import pallas as pl
from jax.experimental.pallas import tpu as pltpu
from jax.experimental.pallas import tpu_sc as plsc

N = 10000
N_PAD = 10240
E = 160000
F_IN = 256
H = 256
G = 64
D2 = H + F_IN
D3 = 2 * H + F_IN
CH = 128
BLK = 128
NSUB = 16
BPT = 80
E_PAD = NSUB * BLK * BPT
NBLK = E_PAD // BLK
GRP = 16
RPT = N_PAD // NSUB
RB = 1280
NRB = N_PAD // RB
EPS = 1e-5


def _sc_agg(hw, gidx, dst_p, ew_p, nc):
    ncpc = nc // 2

    cp = pltpu.CompilerParams()
    if "needs_layout_passes" in pltpu.CompilerParams.__dataclass_fields__:
        cp = dataclasses.replace(cp, needs_layout_passes=False)

    @functools.partial(
        pl.kernel,
        compiler_params=cp,
        out_type=jax.ShapeDtypeStruct((nc * N_PAD, CH), jnp.float32),
        mesh=plsc.VectorSubcoreMesh(core_axis_name="c", subcore_axis_name="s"),
        scratch_types=[
            pltpu.VMEM((GRP, BLK), jnp.int32),
            pltpu.VMEM((GRP, BLK), jnp.int32),
            pltpu.VMEM((GRP, BLK), jnp.float32),
            pltpu.VMEM((BLK, CH), jnp.float32),
            pltpu.VMEM((BLK, CH), jnp.float32),
            pltpu.VMEM_SHARED((N_PAD, CH), jnp.float32),
            pltpu.SemaphoreType.DMA,
            pltpu.SemaphoreType.DMA,
        ],
    )
    def k(hw_hbm, gidx_hbm, dst_hbm, ew_hbm, out_hbm,
          gidx_v, dst_v, ew_v, rows_a, rows_b, acc_sh, sga, sgb):
        core = lax.axis_index("c")
        sid = lax.axis_index("s")
        zero = jnp.zeros((16,), jnp.float32)
        zi = jnp.zeros((16,), jnp.int32)

        def scale(rows_v, b):
            @pl.loop(0, BLK, step=2)
            def _scale(e):
                for kk in range(2):
                    ek = e + kk
                    w = plsc.load_gather(ew_v, [zi + b, zi + ek])
                    for c0 in range(0, CH, 16):
                        rows_v[ek, pl.ds(c0, 16)] = (
                            rows_v[ek, pl.ds(c0, 16)] * w)

        for ci in range(ncpc):
            chunk = core * ncpc + ci

            @pl.loop(0, BLK)
            def _zero_rows(r):
                for c0 in range(0, CH, 16):
                    rows_a[r, pl.ds(c0, 16)] = zero

            for z in range(5):
                pltpu.sync_copy(rows_a,
                                acc_sh.at[pl.ds(sid * RPT + z * BLK, BLK)])
            plsc.subcore_barrier()

            @pl.loop(0, BPT // GRP)
            def _grp(g):
                gb = sid * BPT + g * GRP
                pltpu.sync_copy(
                    gidx_hbm.at[pl.ds(chunk * NBLK + gb, GRP)], gidx_v)
                pltpu.sync_copy(dst_hbm.at[pl.ds(gb, GRP)], dst_v)
                pltpu.sync_copy(ew_hbm.at[pl.ds(gb, GRP)], ew_v)
                pltpu.async_copy(hw_hbm.at[gidx_v.at[0]], rows_a, sga)

                @pl.loop(0, GRP, step=2)
                def _edges(b):
                    pltpu.async_copy(hw_hbm.at[gidx_v.at[b + 1]], rows_b, sgb)
                    pltpu.make_async_copy(hw_hbm.at[gidx_v.at[b]], rows_a,
                                          sga).wait()
                    scale(rows_a, b)
                    pltpu.sync_copy(rows_a, acc_sh.at[dst_v.at[b]], add=True)

                    @pl.when(b + 2 < GRP)
                    def _():
                        pltpu.async_copy(hw_hbm.at[gidx_v.at[b + 2]],
                                         rows_a, sga)

                    pltpu.make_async_copy(hw_hbm.at[gidx_v.at[b + 1]], rows_b,
                                          sgb).wait()
                    scale(rows_b, b + 1)
                    pltpu.sync_copy(rows_b,
                                    acc_sh.at[dst_v.at[b + 1]], add=True)

            plsc.subcore_barrier()
            for z in range(5):
                pltpu.sync_copy(acc_sh.at[pl.ds(sid * RPT + z * BLK, BLK)],
                                out_hbm.at[pl.ds(chunk * N_PAD + sid * RPT
                                                 + z * BLK, BLK)])
            if ci + 1 < ncpc:
                plsc.subcore_barrier()

    return k(hw, gidx, dst_p, ew_p)


_DNUMS = (((1,), (1,)), ((), ()))


def _tc_mm1(x, W1):
    nc = H // CH

    def body(x_ref, w_ref, o_ref):
        o_ref[...] = lax.dot_general(x_ref[...], w_ref[...], _DNUMS,
                                     preferred_element_type=jnp.float32)

    return pl.pallas_call(
        body,
        grid=(nc, NRB),
        in_specs=[pl.BlockSpec((RB, F_IN), lambda i, j: (j, 0)),
                  pl.BlockSpec((CH, F_IN), lambda i, j: (i, 0))],
        out_specs=pl.BlockSpec((RB, CH), lambda i, j: (i * NRB + j, 0)),
        out_shape=jax.ShapeDtypeStruct((nc * N_PAD, CH), jnp.float32),
    )(x, W1)


def _tc_stats(agg, b2d):
    nc = agg.shape[0]

    def body(a_ref, b_ref, o_ref):
        j = pl.program_id(0)

        @pl.when(j == 0)
        def _():
            o_ref[...] = jnp.zeros_like(o_ref)

        valid = (j * RB + lax.broadcasted_iota(jnp.int32, (RB, 1), 0)) < N
        for c in range(nc):
            t = jnp.maximum(a_ref[c] + b_ref[c][None, :], 0.0)
            t = jnp.where(valid, t, 0.0)
            o_ref[0, c] += jnp.sum(t, axis=0)
            o_ref[1, c] += jnp.sum(t * t, axis=0)

    return pl.pallas_call(
        body,
        grid=(NRB,),
        in_specs=[pl.BlockSpec((nc, RB, CH), lambda j: (0, j, 0)),
                  pl.BlockSpec((nc, CH), lambda j: (0, 0))],
        out_specs=pl.BlockSpec((2, nc, CH), lambda j: (0, 0, 0)),
        out_shape=jax.ShapeDtypeStruct((2, nc, CH), jnp.float32),
    )(agg, b2d)


def _affine(s, g2d, be2d):
    mean = s[0] / N
    var = s[1] / N - mean * mean
    a = g2d * lax.rsqrt(var + EPS)
    return a, be2d - mean * a


def _tc_mm2(agg1, b1r, a1, c1, x, W2h, W2x):
    nc_in = H // CH
    nc_out = D2 // CH

    def body(g_ref, b_ref, a_ref, c_ref, x_ref, wh_ref, wx_ref, o_ref):
        acc = lax.dot_general(x_ref[...], wx_ref[...], _DNUMS,
                              preferred_element_type=jnp.float32)
        for c in range(nc_in):
            t = jnp.maximum(g_ref[c] + b_ref[c][None, :], 0.0)
            u = a_ref[c][None, :] * t + c_ref[c][None, :]
            acc += lax.dot_general(u, wh_ref[:, c * CH:(c + 1) * CH], _DNUMS,
                                   preferred_element_type=jnp.float32)
        o_ref[...] = acc

    return pl.pallas_call(
        body,
        grid=(nc_out, NRB),
        in_specs=[pl.BlockSpec((nc_in, RB, CH), lambda i, j: (0, j, 0)),
                  pl.BlockSpec((nc_in, CH), lambda i, j: (0, 0)),
                  pl.BlockSpec((nc_in, CH), lambda i, j: (0, 0)),
                  pl.BlockSpec((nc_in, CH), lambda i, j: (0, 0)),
                  pl.BlockSpec((RB, F_IN), lambda i, j: (j, 0)),
                  pl.BlockSpec((CH, H), lambda i, j: (i, 0)),
                  pl.BlockSpec((CH, F_IN), lambda i, j: (i, 0))],
        out_specs=pl.BlockSpec((RB, CH), lambda i, j: (i * NRB + j, 0)),
        out_shape=jax.ShapeDtypeStruct((nc_out * N_PAD, CH), jnp.float32),
    )(agg1, b1r, a1, c1, x, W2h, W2x)


def _tc_mm3(agg2, b2r, a2, c2, agg1, b1r, a1, c1, W3h2, W3h1):
    nc2 = D2 // CH
    nc1 = H // CH
    nc_out = D3 // CH

    def body(g2_ref, b2_ref, a2_ref, c2_ref, g1_ref, b1_ref, a1_ref, c1_ref,
             wh2_ref, wh1_ref, o_ref):
        acc = None
        for c in range(nc2):
            t = jnp.maximum(g2_ref[c] + b2_ref[c][None, :], 0.0)
            u = a2_ref[c][None, :] * t + c2_ref[c][None, :]
            p = lax.dot_general(u, wh2_ref[:, c * CH:(c + 1) * CH], _DNUMS,
                                preferred_element_type=jnp.float32)
            acc = p if acc is None else acc + p
        for c in range(nc1):
            t = jnp.maximum(g1_ref[c] + b1_ref[c][None, :], 0.0)
            u = a1_ref[c][None, :] * t + c1_ref[c][None, :]
            acc += lax.dot_general(u, wh1_ref[:, c * CH:(c + 1) * CH], _DNUMS,
                                   preferred_element_type=jnp.float32)
        o_ref[...] = acc

    return pl.pallas_call(
        body,
        grid=(nc_out, NRB),
        in_specs=[pl.BlockSpec((nc2, RB, CH), lambda i, j: (0, j, 0)),
                  pl.BlockSpec((nc2, CH), lambda i, j: (0, 0)),
                  pl.BlockSpec((nc2, CH), lambda i, j: (0, 0)),
                  pl.BlockSpec((nc2, CH), lambda i, j: (0, 0)),
                  pl.BlockSpec((nc1, RB, CH), lambda i, j: (0, j, 0)),
                  pl.BlockSpec((nc1, CH), lambda i, j: (0, 0)),
                  pl.BlockSpec((nc1, CH), lambda i, j: (0, 0)),
                  pl.BlockSpec((nc1, CH), lambda i, j: (0, 0)),
                  pl.BlockSpec((CH, D2), lambda i, j: (i, 0)),
                  pl.BlockSpec((CH, H), lambda i, j: (i, 0))],
        out_specs=pl.BlockSpec((RB, CH), lambda i, j: (i * NRB + j, 0)),
        out_shape=jax.ShapeDtypeStruct((nc_out * N_PAD, CH), jnp.float32),
    )(agg2, b2r, a2, c2, agg1, b1r, a1, c1, W3h2, W3h1)


def _tc_pool(agg3, batch3d):
    nc = D3 // CH

    def body(a_ref, bt_ref, seg_ref, cnt_ref, s_ref):
        j = pl.program_id(0)

        @pl.when(j == 0)
        def _():
            seg_ref[...] = jnp.zeros_like(seg_ref)
            cnt_ref[...] = jnp.zeros_like(cnt_ref)
            s_ref[...] = jnp.zeros_like(s_ref)

        b = bt_ref[0, 0, :]
        oh = (b[:, None] == lax.broadcasted_iota(jnp.int32, (RB, G), 1)
              ).astype(jnp.float32)
        cnt_ref[0, :] += jnp.sum(oh, axis=0)
        for c in range(nc):
            blkc = a_ref[c]
            seg_ref[c] += lax.dot_general(oh, blkc, (((0,), (0,)), ((), ())),
                                          preferred_element_type=jnp.float32)
            s_ref[0, c] += jnp.sum(blkc, axis=0)
            s_ref[1, c] += jnp.sum(blkc * blkc, axis=0)

    return pl.pallas_call(
        body,
        grid=(NRB,),
        in_specs=[pl.BlockSpec((nc, RB, CH), lambda j: (0, j, 0)),
                  pl.BlockSpec((1, 1, RB), lambda j: (j, 0, 0))],
        out_specs=[pl.BlockSpec((nc, G, CH), lambda j: (0, 0, 0)),
                   pl.BlockSpec((1, G), lambda j: (0, 0)),
                   pl.BlockSpec((2, nc, CH), lambda j: (0, 0, 0))],
        out_shape=[jax.ShapeDtypeStruct((nc, G, CH), jnp.float32),
                   jax.ShapeDtypeStruct((1, G), jnp.float32),
                   jax.ShapeDtypeStruct((2, nc, CH), jnp.float32)],
    )(agg3, batch3d)


def _tc_final(seg, cnt, s3, b3r, g3r, be3r, g4r, be4r, wlr, bl2):
    nc = D3 // CH

    def body(seg_ref, cnt_ref, s_ref, b3_ref, g3_ref, be3_ref,
             g4_ref, be4_ref, wl_ref, bl_ref, o_ref):
        cnts = jnp.maximum(cnt_ref[0, :], 1.0)
        outv = jnp.zeros((G,), jnp.float32)
        for c in range(nc):
            m0 = s_ref[0, c] / N
            var3 = s_ref[1, c] / N - m0 * m0
            a3 = g3_ref[c] * lax.rsqrt(var3 + EPS)
            pooled = seg_ref[c] / cnts[:, None]
            p = a3[None, :] * (pooled - m0[None, :]) + be3_ref[c][None, :]
            m4 = jnp.mean(p, axis=0)
            v4 = jnp.mean(p * p, axis=0) - m4 * m4
            p4 = (g4_ref[c][None, :] * (p - m4[None, :])
                  * lax.rsqrt(v4 + EPS)[None, :] + be4_ref[c][None, :])
            outv += jnp.sum(p4 * wl_ref[c][None, :], axis=1)
        o_ref[...] = (outv + bl_ref[0, 0])[None, :]

    return pl.pallas_call(
        body,
        in_specs=[pl.BlockSpec(seg.shape, lambda: (0, 0, 0)),
                  pl.BlockSpec(cnt.shape, lambda: (0, 0)),
                  pl.BlockSpec(s3.shape, lambda: (0, 0, 0)),
                  pl.BlockSpec(b3r.shape, lambda: (0, 0)),
                  pl.BlockSpec(g3r.shape, lambda: (0, 0)),
                  pl.BlockSpec(be3r.shape, lambda: (0, 0)),
                  pl.BlockSpec(g4r.shape, lambda: (0, 0)),
                  pl.BlockSpec(be4r.shape, lambda: (0, 0)),
                  pl.BlockSpec(wlr.shape, lambda: (0, 0)),
                  pl.BlockSpec(bl2.shape, lambda: (0, 0))],
        out_specs=pl.BlockSpec((1, G), lambda: (0, 0)),
        out_shape=jax.ShapeDtypeStruct((1, G), jnp.float32),
    )(seg, cnt, s3, b3r, g3r, be3r, g4r, be4r, wlr, bl2)


def kernel(x, edge_index, edge_weight, batch, W1, b1, g1, be1, W2, b2, g2, be2,
           W3, b3, g3, be3, g4, be4, Wl, bl):
    f32 = jnp.float32
    x = x.astype(f32)
    src = edge_index[0].astype(jnp.int32)
    dst = edge_index[1].astype(jnp.int32)
    ew = edge_weight.astype(f32)
    pad = E_PAD - E
    src_p = jnp.concatenate([src, jnp.zeros((pad,), jnp.int32)])
    dst_p = jnp.concatenate([dst, jnp.zeros((pad,), jnp.int32)]).reshape(
        NBLK, BLK)
    ew_p = jnp.concatenate([ew, jnp.zeros((pad,), f32)]).reshape(NBLK, BLK)
    gidx6 = (src_p[None, :]
             + (N_PAD * jnp.arange(6, dtype=jnp.int32))[:, None])
    x = jnp.concatenate([x, jnp.zeros((N_PAD - N, F_IN), f32)])
    batch_p = jnp.concatenate(
        [batch.astype(jnp.int32), jnp.full((N_PAD - N,), G, jnp.int32)])

    hw1 = _tc_mm1(x, W1)
    agg1 = _sc_agg(hw1, gidx6[:2].reshape(2 * NBLK, BLK), dst_p, ew_p,
                   2).reshape(2, N_PAD, CH)
    b1r = b1.reshape(2, CH)
    a1, c1 = _affine(_tc_stats(agg1, b1r), g1.reshape(2, CH),
                     be1.reshape(2, CH))

    hw2 = _tc_mm2(agg1, b1r, a1, c1, x, W2[:, :H], W2[:, H:])
    agg2 = _sc_agg(hw2, gidx6[:4].reshape(4 * NBLK, BLK), dst_p, ew_p,
                   4).reshape(4, N_PAD, CH)
    b2r = b2.reshape(4, CH)
    a2, c2 = _affine(_tc_stats(agg2, b2r), g2.reshape(4, CH),
                     be2.reshape(4, CH))

    hw3 = _tc_mm3(agg2, b2r, a2, c2, agg1, b1r, a1, c1, W3[:, :D2], W3[:, D2:])
    agg3 = _sc_agg(hw3, gidx6.reshape(6 * NBLK, BLK), dst_p, ew_p,
                   6).reshape(6, N_PAD, CH)

    seg, cnt, s3 = _tc_pool(agg3, batch_p.reshape(NRB, 1, RB))
    out = _tc_final(seg, cnt, s3, b3.reshape(6, CH), g3.reshape(6, CH),
                    be3.reshape(6, CH), g4.reshape(6, CH), be4.reshape(6, CH),
                    Wl.reshape(6, CH), bl.reshape(1, 1))
    return out.reshape(G, 1)


# speedup vs baseline: 2.8409x; 1.1131x over previous
import dataclasses
import functools

import jax
import jax.numpy as jnp
from jax import lax
from jax.experimental import pallas as pl
from jax.experimental.pallas import tpu as pltpu
from jax.experimental.pallas import tpu_sc as plsc

N = 10000
N_PAD = 10240
E = 160000
F_IN = 256
H = 256
G = 64
D2 = H + F_IN
D3 = 2 * H + F_IN
CH = 128
BLK = 128
NSUB = 16
BPT = 80
E_PAD = NSUB * BLK * BPT
NBLK = E_PAD // BLK
GRP = 16
RPT = N_PAD // NSUB
RB = 1280
NRB = N_PAD // RB
EPS = 1e-5


def _sc_agg(hw, gidx, dst_p, ew_p, nc):
    ncpc = nc // 2

    cp = pltpu.CompilerParams()
    if "needs_layout_passes" in pltpu.CompilerParams.__dataclass_fields__:
        cp = dataclasses.replace(cp, needs_layout_passes=False)

    @functools.partial(
        pl.kernel,
        compiler_params=cp,
        out_type=jax.ShapeDtypeStruct((nc * N_PAD, CH), jnp.float32),
        mesh=plsc.VectorSubcoreMesh(core_axis_name="c", subcore_axis_name="s"),
        scratch_types=[
            pltpu.VMEM((GRP, BLK), jnp.int32),
            pltpu.VMEM((GRP, BLK), jnp.int32),
            pltpu.VMEM((GRP * BLK,), jnp.float32),
            pltpu.VMEM((BLK, CH), jnp.float32),
            pltpu.VMEM((BLK, CH), jnp.float32),
            pltpu.VMEM_SHARED((N_PAD, CH), jnp.float32),
            pltpu.SemaphoreType.DMA,
            pltpu.SemaphoreType.DMA,
        ],
    )
    def k(hw_hbm, gidx_hbm, dst_hbm, ew_hbm, out_hbm,
          gidx_v, dst_v, ew_v, rows_a, rows_b, acc_sh, sga, sgb):
        core = lax.axis_index("c")
        sid = lax.axis_index("s")
        zero = jnp.zeros((16,), jnp.float32)
        zi = jnp.zeros((16,), jnp.int32)

        def scale(rows_v, b):
            base_v = zi + b * BLK

            @functools.partial(plsc.parallel_loop, 0, BLK, unroll=4)
            def _scale(e):
                w = plsc.load_gather(ew_v, [base_v + e])
                for c0 in range(0, CH, 16):
                    rows_v[e, pl.ds(c0, 16)] = rows_v[e, pl.ds(c0, 16)] * w

        for ci in range(ncpc):
            chunk = core * ncpc + ci

            @pl.loop(0, BLK)
            def _zero_rows(r):
                for c0 in range(0, CH, 16):
                    rows_a[r, pl.ds(c0, 16)] = zero

            for z in range(5):
                pltpu.sync_copy(rows_a,
                                acc_sh.at[pl.ds(sid * RPT + z * BLK, BLK)])
            plsc.subcore_barrier()

            @pl.loop(0, BPT // GRP)
            def _grp(g):
                gb = sid * BPT + g * GRP
                pltpu.sync_copy(
                    gidx_hbm.at[pl.ds(chunk * NBLK + gb, GRP)], gidx_v)
                pltpu.sync_copy(dst_hbm.at[pl.ds(gb, GRP)], dst_v)
                pltpu.sync_copy(ew_hbm.at[pl.ds(gb * BLK, GRP * BLK)], ew_v)
                pltpu.async_copy(hw_hbm.at[gidx_v.at[0]], rows_a, sga)

                @pl.loop(0, GRP, step=2)
                def _edges(b):
                    pltpu.async_copy(hw_hbm.at[gidx_v.at[b + 1]], rows_b, sgb)
                    pltpu.make_async_copy(hw_hbm.at[gidx_v.at[b]], rows_a,
                                          sga).wait()
                    scale(rows_a, b)
                    pltpu.sync_copy(rows_a, acc_sh.at[dst_v.at[b]], add=True)

                    @pl.when(b + 2 < GRP)
                    def _():
                        pltpu.async_copy(hw_hbm.at[gidx_v.at[b + 2]],
                                         rows_a, sga)

                    pltpu.make_async_copy(hw_hbm.at[gidx_v.at[b + 1]], rows_b,
                                          sgb).wait()
                    scale(rows_b, b + 1)
                    pltpu.sync_copy(rows_b,
                                    acc_sh.at[dst_v.at[b + 1]], add=True)

            plsc.subcore_barrier()
            for z in range(5):
                pltpu.sync_copy(acc_sh.at[pl.ds(sid * RPT + z * BLK, BLK)],
                                out_hbm.at[pl.ds(chunk * N_PAD + sid * RPT
                                                 + z * BLK, BLK)])
            if ci + 1 < ncpc:
                plsc.subcore_barrier()

    return k(hw, gidx, dst_p, ew_p)


_DNUMS = (((1,), (1,)), ((), ()))


def _tc_mm1(x, W1):
    nc = H // CH

    def body(x_ref, w_ref, o_ref):
        o_ref[...] = lax.dot_general(x_ref[...], w_ref[...], _DNUMS,
                                     preferred_element_type=jnp.float32)

    return pl.pallas_call(
        body,
        grid=(nc, NRB),
        in_specs=[pl.BlockSpec((RB, F_IN), lambda i, j: (j, 0)),
                  pl.BlockSpec((CH, F_IN), lambda i, j: (i, 0))],
        out_specs=pl.BlockSpec((RB, CH), lambda i, j: (i * NRB + j, 0)),
        out_shape=jax.ShapeDtypeStruct((nc * N_PAD, CH), jnp.float32),
    )(x, W1)


def _tc_stats(agg, b2d):
    nc = agg.shape[0]

    def body(a_ref, b_ref, o_ref):
        j = pl.program_id(0)

        @pl.when(j == 0)
        def _():
            o_ref[...] = jnp.zeros_like(o_ref)

        valid = (j * RB + lax.broadcasted_iota(jnp.int32, (RB, 1), 0)) < N
        for c in range(nc):
            t = jnp.maximum(a_ref[c] + b_ref[c][None, :], 0.0)
            t = jnp.where(valid, t, 0.0)
            o_ref[0, c] += jnp.sum(t, axis=0)
            o_ref[1, c] += jnp.sum(t * t, axis=0)

    return pl.pallas_call(
        body,
        grid=(NRB,),
        in_specs=[pl.BlockSpec((nc, RB, CH), lambda j: (0, j, 0)),
                  pl.BlockSpec((nc, CH), lambda j: (0, 0))],
        out_specs=pl.BlockSpec((2, nc, CH), lambda j: (0, 0, 0)),
        out_shape=jax.ShapeDtypeStruct((2, nc, CH), jnp.float32),
    )(agg, b2d)


def _affine(s, g2d, be2d):
    mean = s[0] / N
    var = s[1] / N - mean * mean
    a = g2d * lax.rsqrt(var + EPS)
    return a, be2d - mean * a


def _tc_mm2(agg1, b1r, a1, c1, x, W2h, W2x):
    nc_in = H // CH
    nc_out = D2 // CH

    def body(g_ref, b_ref, a_ref, c_ref, x_ref, wh_ref, wx_ref, o_ref):
        acc = lax.dot_general(x_ref[...], wx_ref[...], _DNUMS,
                              preferred_element_type=jnp.float32)
        for c in range(nc_in):
            t = jnp.maximum(g_ref[c] + b_ref[c][None, :], 0.0)
            u = a_ref[c][None, :] * t + c_ref[c][None, :]
            acc += lax.dot_general(u, wh_ref[:, c * CH:(c + 1) * CH], _DNUMS,
                                   preferred_element_type=jnp.float32)
        o_ref[...] = acc

    return pl.pallas_call(
        body,
        grid=(nc_out, NRB),
        in_specs=[pl.BlockSpec((nc_in, RB, CH), lambda i, j: (0, j, 0)),
                  pl.BlockSpec((nc_in, CH), lambda i, j: (0, 0)),
                  pl.BlockSpec((nc_in, CH), lambda i, j: (0, 0)),
                  pl.BlockSpec((nc_in, CH), lambda i, j: (0, 0)),
                  pl.BlockSpec((RB, F_IN), lambda i, j: (j, 0)),
                  pl.BlockSpec((CH, H), lambda i, j: (i, 0)),
                  pl.BlockSpec((CH, F_IN), lambda i, j: (i, 0))],
        out_specs=pl.BlockSpec((RB, CH), lambda i, j: (i * NRB + j, 0)),
        out_shape=jax.ShapeDtypeStruct((nc_out * N_PAD, CH), jnp.float32),
    )(agg1, b1r, a1, c1, x, W2h, W2x)


def _tc_mm3(agg2, b2r, a2, c2, agg1, b1r, a1, c1, W3h2, W3h1):
    nc2 = D2 // CH
    nc1 = H // CH
    nc_out = D3 // CH

    def body(g2_ref, b2_ref, a2_ref, c2_ref, g1_ref, b1_ref, a1_ref, c1_ref,
             wh2_ref, wh1_ref, o_ref):
        acc = None
        for c in range(nc2):
            t = jnp.maximum(g2_ref[c] + b2_ref[c][None, :], 0.0)
            u = a2_ref[c][None, :] * t + c2_ref[c][None, :]
            p = lax.dot_general(u, wh2_ref[:, c * CH:(c + 1) * CH], _DNUMS,
                                preferred_element_type=jnp.float32)
            acc = p if acc is None else acc + p
        for c in range(nc1):
            t = jnp.maximum(g1_ref[c] + b1_ref[c][None, :], 0.0)
            u = a1_ref[c][None, :] * t + c1_ref[c][None, :]
            acc += lax.dot_general(u, wh1_ref[:, c * CH:(c + 1) * CH], _DNUMS,
                                   preferred_element_type=jnp.float32)
        o_ref[...] = acc

    return pl.pallas_call(
        body,
        grid=(nc_out, NRB),
        in_specs=[pl.BlockSpec((nc2, RB, CH), lambda i, j: (0, j, 0)),
                  pl.BlockSpec((nc2, CH), lambda i, j: (0, 0)),
                  pl.BlockSpec((nc2, CH), lambda i, j: (0, 0)),
                  pl.BlockSpec((nc2, CH), lambda i, j: (0, 0)),
                  pl.BlockSpec((nc1, RB, CH), lambda i, j: (0, j, 0)),
                  pl.BlockSpec((nc1, CH), lambda i, j: (0, 0)),
                  pl.BlockSpec((nc1, CH), lambda i, j: (0, 0)),
                  pl.BlockSpec((nc1, CH), lambda i, j: (0, 0)),
                  pl.BlockSpec((CH, D2), lambda i, j: (i, 0)),
                  pl.BlockSpec((CH, H), lambda i, j: (i, 0))],
        out_specs=pl.BlockSpec((RB, CH), lambda i, j: (i * NRB + j, 0)),
        out_shape=jax.ShapeDtypeStruct((nc_out * N_PAD, CH), jnp.float32),
    )(agg2, b2r, a2, c2, agg1, b1r, a1, c1, W3h2, W3h1)


def _tc_pool(agg3, batch3d):
    nc = D3 // CH

    def body(a_ref, bt_ref, seg_ref, cnt_ref, s_ref):
        j = pl.program_id(0)

        @pl.when(j == 0)
        def _():
            seg_ref[...] = jnp.zeros_like(seg_ref)
            cnt_ref[...] = jnp.zeros_like(cnt_ref)
            s_ref[...] = jnp.zeros_like(s_ref)

        b = bt_ref[0, 0, :]
        oh = (b[:, None] == lax.broadcasted_iota(jnp.int32, (RB, G), 1)
              ).astype(jnp.float32)
        cnt_ref[0, :] += jnp.sum(oh, axis=0)
        for c in range(nc):
            blkc = a_ref[c]
            seg_ref[c] += lax.dot_general(oh, blkc, (((0,), (0,)), ((), ())),
                                          preferred_element_type=jnp.float32)
            s_ref[0, c] += jnp.sum(blkc, axis=0)
            s_ref[1, c] += jnp.sum(blkc * blkc, axis=0)

    return pl.pallas_call(
        body,
        grid=(NRB,),
        in_specs=[pl.BlockSpec((nc, RB, CH), lambda j: (0, j, 0)),
                  pl.BlockSpec((1, 1, RB), lambda j: (j, 0, 0))],
        out_specs=[pl.BlockSpec((nc, G, CH), lambda j: (0, 0, 0)),
                   pl.BlockSpec((1, G), lambda j: (0, 0)),
                   pl.BlockSpec((2, nc, CH), lambda j: (0, 0, 0))],
        out_shape=[jax.ShapeDtypeStruct((nc, G, CH), jnp.float32),
                   jax.ShapeDtypeStruct((1, G), jnp.float32),
                   jax.ShapeDtypeStruct((2, nc, CH), jnp.float32)],
    )(agg3, batch3d)


def _tc_final(seg, cnt, s3, b3r, g3r, be3r, g4r, be4r, wlr, bl2):
    nc = D3 // CH

    def body(seg_ref, cnt_ref, s_ref, b3_ref, g3_ref, be3_ref,
             g4_ref, be4_ref, wl_ref, bl_ref, o_ref):
        cnts = jnp.maximum(cnt_ref[0, :], 1.0)
        outv = jnp.zeros((G,), jnp.float32)
        for c in range(nc):
            m0 = s_ref[0, c] / N
            var3 = s_ref[1, c] / N - m0 * m0
            a3 = g3_ref[c] * lax.rsqrt(var3 + EPS)
            pooled = seg_ref[c] / cnts[:, None]
            p = a3[None, :] * (pooled - m0[None, :]) + be3_ref[c][None, :]
            m4 = jnp.mean(p, axis=0)
            v4 = jnp.mean(p * p, axis=0) - m4 * m4
            p4 = (g4_ref[c][None, :] * (p - m4[None, :])
                  * lax.rsqrt(v4 + EPS)[None, :] + be4_ref[c][None, :])
            outv += jnp.sum(p4 * wl_ref[c][None, :], axis=1)
        o_ref[...] = (outv + bl_ref[0, 0])[None, :]

    return pl.pallas_call(
        body,
        in_specs=[pl.BlockSpec(seg.shape, lambda: (0, 0, 0)),
                  pl.BlockSpec(cnt.shape, lambda: (0, 0)),
                  pl.BlockSpec(s3.shape, lambda: (0, 0, 0)),
                  pl.BlockSpec(b3r.shape, lambda: (0, 0)),
                  pl.BlockSpec(g3r.shape, lambda: (0, 0)),
                  pl.BlockSpec(be3r.shape, lambda: (0, 0)),
                  pl.BlockSpec(g4r.shape, lambda: (0, 0)),
                  pl.BlockSpec(be4r.shape, lambda: (0, 0)),
                  pl.BlockSpec(wlr.shape, lambda: (0, 0)),
                  pl.BlockSpec(bl2.shape, lambda: (0, 0))],
        out_specs=pl.BlockSpec((1, G), lambda: (0, 0)),
        out_shape=jax.ShapeDtypeStruct((1, G), jnp.float32),
    )(seg, cnt, s3, b3r, g3r, be3r, g4r, be4r, wlr, bl2)


def kernel(x, edge_index, edge_weight, batch, W1, b1, g1, be1, W2, b2, g2, be2,
           W3, b3, g3, be3, g4, be4, Wl, bl):
    f32 = jnp.float32
    x = x.astype(f32)
    src = edge_index[0].astype(jnp.int32)
    dst = edge_index[1].astype(jnp.int32)
    ew = edge_weight.astype(f32)
    pad = E_PAD - E
    src_p = jnp.concatenate([src, jnp.zeros((pad,), jnp.int32)])
    dst_p = jnp.concatenate([dst, jnp.zeros((pad,), jnp.int32)]).reshape(
        NBLK, BLK)
    ew_p = jnp.concatenate([ew, jnp.zeros((pad,), f32)])
    gidx6 = (src_p[None, :]
             + (N_PAD * jnp.arange(6, dtype=jnp.int32))[:, None])
    x = jnp.concatenate([x, jnp.zeros((N_PAD - N, F_IN), f32)])
    batch_p = jnp.concatenate(
        [batch.astype(jnp.int32), jnp.full((N_PAD - N,), G, jnp.int32)])

    hw1 = _tc_mm1(x, W1)
    agg1 = _sc_agg(hw1, gidx6[:2].reshape(2 * NBLK, BLK), dst_p, ew_p,
                   2).reshape(2, N_PAD, CH)
    b1r = b1.reshape(2, CH)
    a1, c1 = _affine(_tc_stats(agg1, b1r), g1.reshape(2, CH),
                     be1.reshape(2, CH))

    hw2 = _tc_mm2(agg1, b1r, a1, c1, x, W2[:, :H], W2[:, H:])
    agg2 = _sc_agg(hw2, gidx6[:4].reshape(4 * NBLK, BLK), dst_p, ew_p,
                   4).reshape(4, N_PAD, CH)
    b2r = b2.reshape(4, CH)
    a2, c2 = _affine(_tc_stats(agg2, b2r), g2.reshape(4, CH),
                     be2.reshape(4, CH))

    hw3 = _tc_mm3(agg2, b2r, a2, c2, agg1, b1r, a1, c1, W3[:, :D2], W3[:, D2:])
    agg3 = _sc_agg(hw3, gidx6.reshape(6 * NBLK, BLK), dst_p, ew_p,
                   6).reshape(6, N_PAD, CH)

    seg, cnt, s3 = _tc_pool(agg3, batch_p.reshape(NRB, 1, RB))
    out = _tc_final(seg, cnt, s3, b3.reshape(6, CH), g3.reshape(6, CH),
                    be3.reshape(6, CH), g4.reshape(6, CH), be4.reshape(6, CH),
                    Wl.reshape(6, CH), bl.reshape(1, 1))
    return out.reshape(G, 1)
